# Initial kernel scaffold; baseline (speedup 1.0000x reference)
import jax
import jax.numpy as jnp
from jax.experimental import pallas as pl


def kernel(scores, boxes, score_covs, box_covs):
    raise NotImplementedError("write your pallas kernel here")


# speedup vs baseline: 4.6276x; 4.6276x over previous
import functools

import jax
import jax.numpy as jnp
from jax import lax
from jax.experimental import pallas as pl
from jax.experimental.pallas import tpu as pltpu

_NUM_CLASSES = 20
_SCORE_THRESH = 0.95
_NMS_THRESH = 0.5
_TOPK = 100
_IMG_W = 1024.0
_IMG_H = 1024.0

_LANES = 128
_SUB = 8
_BLK = _SUB * _LANES


def _nms_body(m, nblocks,
              sc_in, x1_in, y1_in, x2_in, y2_in,
              c0_in, c1_in, c2_in, c3_in, scov_in,
              outv, outc, msc):
    neg = jnp.float32(-jnp.inf)
    lane_i = lax.broadcasted_iota(jnp.int32, (1, _LANES), 1)
    flat_i = (lax.broadcasted_iota(jnp.int32, (_SUB, _LANES), 0) * _LANES
              + lax.broadcasted_iota(jnp.int32, (_SUB, _LANES), 1))

    def pre(r, bm):
        blk = sc_in[pl.ds(r * _SUB, _SUB), :]
        valid = (blk > _SCORE_THRESH) & ((r * _BLK + flat_i) < m)
        mb = jnp.where(valid, blk, neg)
        msc[pl.ds(r * _SUB, _SUB), :] = mb
        return jnp.where(lane_i == r, jnp.max(mb), bm)

    bm0 = lax.fori_loop(0, nblocks, pre, jnp.full((1, _LANES), neg))

    def ext(ref, row, lane):
        rowv = ref[pl.ds(row, 1), :]
        return jnp.sum(jnp.where(lane_i == lane, rowv, 0.0))

    zrow = jnp.zeros((1, _LANES), jnp.float32)
    carry0 = (bm0, jnp.int32(0),
              zrow, zrow, zrow, zrow, zrow,
              jnp.full((1, _LANES), -1, jnp.int32),
              zrow, zrow, zrow, zrow, zrow,
              zrow, zrow, zrow, zrow, zrow,
              jnp.full((1, _LANES), -1, jnp.int32))

    def cond(c):
        bm, cnt = c[0], c[1]
        return (cnt < _TOPK) & (jnp.max(bm) > neg)

    def body(c):
        (bm, cnt, kx1, ky1, kx2, ky2, ka, kc,
         ox1, oy1, ox2, oy2, osc, oc0, oc1, oc2, oc3, oscov, ocls) = c
        best = jnp.max(bm)
        b = jnp.min(jnp.where(bm == best, lane_i, jnp.int32(1 << 30)))
        blk = msc[pl.ds(b * _SUB, _SUB), :]
        wi = jnp.min(jnp.where(blk == best, flat_i, jnp.int32(1 << 30)))
        i = b * _BLK + wi
        row = b * _SUB + wi // _LANES
        lane = wi % _LANES
        x1w = jnp.clip(ext(x1_in, row, lane), 0.0, _IMG_W)
        y1w = jnp.clip(ext(y1_in, row, lane), 0.0, _IMG_H)
        x2w = jnp.clip(ext(x2_in, row, lane), 0.0, _IMG_W)
        y2w = jnp.clip(ext(y2_in, row, lane), 0.0, _IMG_H)
        aw = jnp.maximum(x2w - x1w, 0.0) * jnp.maximum(y2w - y1w, 0.0)
        cw = i % _NUM_CLASSES
        iw = jnp.maximum(jnp.minimum(kx2, x2w) - jnp.maximum(kx1, x1w), 0.0)
        ih = jnp.maximum(jnp.minimum(ky2, y2w) - jnp.maximum(ky1, y1w), 0.0)
        inter = iw * ih
        iou = inter / jnp.maximum(ka + aw - inter, 1e-9)
        sup = jnp.any((iou > _NMS_THRESH) & (kc == cw))
        take = jnp.logical_not(sup)
        sel = (lane_i == cnt) & take
        kx1 = jnp.where(sel, x1w, kx1)
        ky1 = jnp.where(sel, y1w, ky1)
        kx2 = jnp.where(sel, x2w, kx2)
        ky2 = jnp.where(sel, y2w, ky2)
        ka = jnp.where(sel, aw, ka)
        kc = jnp.where(sel, cw, kc)
        ox1 = jnp.where(sel, x1w, ox1)
        oy1 = jnp.where(sel, y1w, oy1)
        ox2 = jnp.where(sel, x2w, ox2)
        oy2 = jnp.where(sel, y2w, oy2)
        osc = jnp.where(sel, best, osc)
        oc0 = jnp.where(sel, ext(c0_in, row, lane), oc0)
        oc1 = jnp.where(sel, ext(c1_in, row, lane), oc1)
        oc2 = jnp.where(sel, ext(c2_in, row, lane), oc2)
        oc3 = jnp.where(sel, ext(c3_in, row, lane), oc3)
        oscov = jnp.where(sel, ext(scov_in, row, lane), oscov)
        ocls = jnp.where(sel, cw, ocls)
        cnt = cnt + jnp.where(take, jnp.int32(1), jnp.int32(0))
        blk2 = jnp.where(flat_i == wi, neg, blk)
        msc[pl.ds(b * _SUB, _SUB), :] = blk2
        bm = jnp.where(lane_i == b, jnp.max(blk2), bm)
        return (bm, cnt, kx1, ky1, kx2, ky2, ka, kc,
                ox1, oy1, ox2, oy2, osc, oc0, oc1, oc2, oc3, oscov, ocls)

    c = lax.while_loop(cond, body, carry0)
    (ox1, oy1, ox2, oy2, osc, oc0, oc1, oc2, oc3, oscov, ocls) = c[8:]
    outv[...] = jnp.concatenate(
        [ox1, oy1, ox2, oy2, osc, oc0, oc1, oc2, oc3, oscov,
         jnp.zeros((6, _LANES), jnp.float32)], axis=0)
    outc[...] = jnp.concatenate(
        [ocls, jnp.zeros((7, _LANES), jnp.int32)], axis=0)


def kernel(scores, boxes, score_covs, box_covs):
    n, kp1 = scores.shape
    k = kp1 - 1
    m = n * k
    nblocks = (m + _BLK - 1) // _BLK
    rows = nblocks * _SUB
    pad = rows * _LANES - m

    def lay(x):
        return jnp.pad(x, (0, pad)).reshape(rows, _LANES)

    sc = lay(scores[:, :k].reshape(-1))
    bx = boxes.reshape(-1, 4)
    cv = box_covs.reshape(-1, 4)
    args = (sc, lay(bx[:, 0]), lay(bx[:, 1]), lay(bx[:, 2]), lay(bx[:, 3]),
            lay(cv[:, 0]), lay(cv[:, 1]), lay(cv[:, 2]), lay(cv[:, 3]),
            lay(score_covs[:, :k].reshape(-1)))

    outv, outc = pl.pallas_call(
        functools.partial(_nms_body, m, nblocks),
        out_shape=(jax.ShapeDtypeStruct((16, _LANES), jnp.float32),
                   jax.ShapeDtypeStruct((8, _LANES), jnp.int32)),
        scratch_shapes=[pltpu.VMEM((rows, _LANES), jnp.float32)],
    )(*args)

    out = outv[:10, :_TOPK].T
    classes = outc[0, :_TOPK].astype(jnp.int64)
    return (out, classes)
